# baseline (speedup 1.0000x reference)
import functools

import numpy as np
import jax
import jax.numpy as jnp
from jax import lax
from jax.experimental import pallas as pl
from jax.experimental.pallas import tpu as pltpu
from jax.experimental.pallas import tpu_sc as plsc

_NC = 2
_NS = 16
_L = 16
_CH = 128


_MC = 4
_ME = _MC * _CH


def _sc_edge_pass(a_tab, b_tab, c8, ei3, with_counts):
    n = a_tab.shape[0]
    e8 = c8.shape[0]
    e = e8 * 8
    nw = _NC * _NS
    nmac = e // _ME
    mep = _ME // 8
    zr = 640
    npad = zr * _NS
    per_w = (nmac + nw - 1) // nw
    per_w += per_w % 2

    out_type = [
        jax.ShapeDtypeStruct((e8, 8 * _L), jnp.float32),
        jax.ShapeDtypeStruct((_NC * npad, _L), jnp.float32),
    ]

    def bufset():
        return [
            pltpu.VMEM((2, _MC, _CH), jnp.int32),
            pltpu.VMEM((_ME, _L), jnp.float32),
            pltpu.VMEM((_ME, _L), jnp.float32),
            pltpu.VMEM((mep, 8 * _L), jnp.float32),
            pltpu.VMEM((_ME, _L), jnp.float32),
            pltpu.VMEM((mep, 8 * _L), jnp.float32),
            pltpu.SemaphoreType.DMA,
            pltpu.SemaphoreType.DMA,
            pltpu.SemaphoreType.DMA,
        ]

    scratch = bufset() + bufset() + [
        pltpu.VMEM((zr, _L), jnp.float32),
        pltpu.VMEM_SHARED((npad, _L), jnp.float32),
    ]
    if with_counts:
        out_type.append(jax.ShapeDtypeStruct((_NC * npad, _L), jnp.float32))
        scratch.append(pltpu.VMEM((_CH, _L), jnp.float32))
        scratch.append(pltpu.VMEM_SHARED((npad, _L), jnp.float32))

    mesh = plsc.VectorSubcoreMesh(core_axis_name="c", subcore_axis_name="s")

    @functools.partial(
        pl.kernel, out_type=tuple(out_type), mesh=mesh,
        scratch_types=scratch,
        compiler_params=pltpu.CompilerParams(use_tc_tiling_on_sc=False))
    def sc_kernel(a_hbm, b_hbm, c_hbm, ei_hbm, eo_hbm, agg_hbm, *rest):
        if with_counts:
            cnt_hbm = rest[0]
            rest = rest[1:]
        s0 = rest[0:9]
        s1 = rest[9:18]
        zv, agg_sp = rest[18], rest[19]
        if with_counts:
            onesv, cnt_sp = rest[20], rest[21]
        cid = lax.axis_index("c")
        sid = lax.axis_index("s")
        wid = sid * _NC + cid

        def issue_pre(m, S):
            idxb, _, _, cv8, _, _, sem_pre, _, _ = S
            mb = pl.multiple_of(m * _MC, _MC)
            pltpu.async_copy(ei_hbm.at[:, pl.ds(mb, _MC)], idxb, sem_pre)
            mb8 = pl.multiple_of(m * mep, mep)
            pltpu.async_copy(c_hbm.at[pl.ds(mb8, mep)], cv8, sem_pre)

        def wait_pre(S):
            idxb, _, _, cv8, _, _, sem_pre, _, _ = S
            pltpu.make_async_copy(ei_hbm.at[:, pl.ds(0, _MC)], idxb,
                                  sem_pre).wait()
            pltpu.make_async_copy(c_hbm.at[pl.ds(0, mep)], cv8,
                                  sem_pre).wait()

        def issue_gath(S):
            idxb, av, bv, _, _, _, _, sem_g, _ = S
            for j in range(_MC):
                pltpu.async_copy(a_hbm.at[idxb.at[0, j]],
                                 av.at[pl.ds(j * _CH, _CH)], sem_g)
                pltpu.async_copy(b_hbm.at[idxb.at[1, j]],
                                 bv.at[pl.ds(j * _CH, _CH)], sem_g)

        def wait_gath(S):
            idxb, av, bv, _, _, _, _, sem_g, _ = S
            for j in range(_MC):
                pltpu.make_async_copy(a_hbm.at[idxb.at[0, j]],
                                      av.at[pl.ds(j * _CH, _CH)],
                                      sem_g).wait()
                pltpu.make_async_copy(b_hbm.at[idxb.at[1, j]],
                                      bv.at[pl.ds(j * _CH, _CH)],
                                      sem_g).wait()

        def drain_out(S):
            _, _, _, _, _, eov8, _, _, sem_out = S
            pltpu.make_async_copy(eov8, eo_hbm.at[pl.ds(0, mep)],
                                  sem_out).wait()

        def run_macro(m, S):
            idxb, av, bv, cv8, eov, eov8, _, _, sem_out = S

            @pl.loop(0, mep)
            def _rows(i):
                e0 = i * 8
                for j in range(8):
                    v = (av[e0 + j] + bv[e0 + j]
                         + cv8[i, pl.ds(j * _L, _L)])
                    eov[e0 + j] = v
                    eov8[i, pl.ds(j * _L, _L)] = v

            mb8 = pl.multiple_of(m * mep, mep)
            pltpu.async_copy(eov8, eo_hbm.at[pl.ds(mb8, mep)], sem_out)
            for j in range(_MC):
                pltpu.sync_copy(eov.at[pl.ds(j * _CH, _CH)],
                                agg_sp.at[idxb.at[1, j]], add=True)
                if with_counts:
                    pltpu.sync_copy(onesv, cnt_sp.at[idxb.at[1, j]],
                                    add=True)

        @pl.loop(0, zr)
        def _zfill(j):
            zv[j] = jnp.zeros((_L,), jnp.float32)

        zoff = pl.multiple_of(sid * zr, zr)
        pltpu.sync_copy(zv, agg_sp.at[pl.ds(zoff, zr)])
        if with_counts:
            @pl.loop(0, _CH)
            def _ofill(j):
                onesv[j] = jnp.ones((_L,), jnp.float32)
            pltpu.sync_copy(zv, cnt_sp.at[pl.ds(zoff, zr)])
        plsc.subcore_barrier()

        issue_pre(wid, s0)
        issue_pre(wid + nw, s1)
        wait_pre(s0)
        issue_gath(s0)

        @pl.loop(0, per_w, step=2)
        def _pipe(k):
            for off, cur, nxt in ((0, s0, s1), (1, s1, s0)):
                kk = k + off
                m_cur = wid + kk * nw
                m_nxt = wid + (kk + 1) * nw
                m_pre = wid + (kk + 2) * nw

                @pl.when(m_cur < nmac)
                def _():
                    wait_gath(cur)

                    @pl.when(m_nxt < nmac)
                    def _():
                        wait_pre(nxt)
                        issue_gath(nxt)

                    @pl.when(kk >= 2)
                    def _():
                        drain_out(cur)

                    run_macro(m_cur, cur)

                    @pl.when(m_pre < nmac)
                    def _():
                        issue_pre(m_pre, cur)

        drain_out(s0)
        drain_out(s1)

        plsc.subcore_barrier()
        osl = pl.multiple_of(sid * zr, zr)
        ohb = pl.multiple_of(cid * npad + sid * zr, zr)
        pltpu.sync_copy(agg_sp.at[pl.ds(osl, zr)], agg_hbm.at[pl.ds(ohb, zr)])
        if with_counts:
            pltpu.sync_copy(cnt_sp.at[pl.ds(osl, zr)],
                            cnt_hbm.at[pl.ds(ohb, zr)])

    outs = sc_kernel(a_tab, b_tab, c8, ei3)
    if with_counts:
        eo8, agg, cnt = outs
        return eo8, (agg[:n], agg[npad:npad + n]), (cnt[:n], cnt[npad:npad + n])
    eo8, agg = outs
    return eo8, (agg[:n], agg[npad:npad + n]), None


def _full(shape):
    return pl.BlockSpec(shape, lambda i: (0, 0))


def _pre(h_v, he8, ws, wd, k0, be_t, grid=25):
    n, dv = h_v.shape
    de = ws.shape[1]
    e8 = he8.shape[0]
    bn = n // grid
    be_rows = e8 // grid

    def body(hv_ref, he_ref, ws_ref, wd_ref, k_ref, be_ref,
             a_ref, b_ref, c_ref):
        hv = hv_ref[...]
        a_ref[...] = jnp.dot(hv, ws_ref[...], preferred_element_type=jnp.float32)
        b_ref[...] = jnp.dot(hv, wd_ref[...], preferred_element_type=jnp.float32)
        c_ref[...] = (jnp.dot(he_ref[...], k_ref[...],
                              preferred_element_type=jnp.float32) + be_ref[...])

    return pl.pallas_call(
        body,
        grid=(grid,),
        in_specs=[pl.BlockSpec((bn, dv), lambda i: (i, 0)),
                  pl.BlockSpec((be_rows, 128), lambda i: (i, 0)),
                  _full((dv, de)), _full((dv, de)),
                  _full((128, 128)), _full((1, 128))],
        out_specs=[pl.BlockSpec((bn, de), lambda i: (i, 0)),
                   pl.BlockSpec((bn, de), lambda i: (i, 0)),
                   pl.BlockSpec((be_rows, 128), lambda i: (i, 0))],
        out_shape=[jax.ShapeDtypeStruct((n, de), jnp.float32),
                   jax.ShapeDtypeStruct((n, de), jnp.float32),
                   jax.ShapeDtypeStruct((e8, 128), jnp.float32)],
    )(h_v, he8, ws, wd, k0, be_t)


def _post(eo8, he8, h_v, agg0, agg1, cnt0, cnt1, m_avg, g_e, b_e,
          wvh, wvm, bv, g_v, b_v, nxt=None, grid=25):
    e8 = eo8.shape[0]
    n, dv = h_v.shape
    de = agg0.shape[1]
    be_rows = e8 // grid
    bn = n // grid
    has_next = nxt is not None

    def body(eo_ref, he_ref, hv_ref, a0_ref, a1_ref, c0_ref, c1_ref,
             m_ref, ge_ref, be_ref, wvh_ref, wvm_ref, bv_ref, gv_ref,
             bv2_ref, *rest):
        if has_next:
            (kn_ref, ben_ref, wsn_ref, wdn_ref,
             hen_ref, hvn_ref, cn_ref, an_ref, bn_ref) = rest
        else:
            hen_ref, hvn_ref = rest
        r = jnp.maximum(eo_ref[...], 0.0)
        mavg = m_ref[...]
        mu = jnp.dot(r, mavg, preferred_element_type=jnp.float32)
        q = r - mu
        var = jnp.dot(q * q, mavg, preferred_element_type=jnp.float32)
        ln = q * lax.rsqrt(var + 1e-5) * ge_ref[...] + be_ref[...]
        hen = he_ref[...] + ln
        hen_ref[...] = hen
        aggt = a0_ref[...] + a1_ref[...]
        cntt = c0_ref[...] + c1_ref[...]
        mean = aggt / jnp.maximum(cntt, 1.0)
        hv = hv_ref[...]
        v = (jnp.dot(hv, wvh_ref[...], preferred_element_type=jnp.float32)
             + jnp.dot(mean, wvm_ref[...], preferred_element_type=jnp.float32)
             + bv_ref[...])
        v = jnp.maximum(v, 0.0)
        mu2 = jnp.mean(v, axis=-1, keepdims=True)
        q2 = v - mu2
        var2 = jnp.mean(q2 * q2, axis=-1, keepdims=True)
        ln2 = q2 * lax.rsqrt(var2 + 1e-5) * gv_ref[...] + bv2_ref[...]
        hvn = hv + ln2
        hvn_ref[...] = hvn
        if has_next:
            cn_ref[...] = (jnp.dot(hen, kn_ref[...],
                                   preferred_element_type=jnp.float32)
                           + ben_ref[...])
            an_ref[...] = jnp.dot(hvn, wsn_ref[...],
                                  preferred_element_type=jnp.float32)
            bn_ref[...] = jnp.dot(hvn, wdn_ref[...],
                                  preferred_element_type=jnp.float32)

    ebs = pl.BlockSpec((be_rows, 128), lambda i: (i, 0))
    vbs = pl.BlockSpec((bn, dv), lambda i: (i, 0))
    sbs = pl.BlockSpec((bn, de), lambda i: (i, 0))
    in_specs = [ebs, ebs, vbs, sbs, sbs, sbs, sbs,
                _full((128, 128)), _full((1, 128)), _full((1, 128)),
                _full((dv, dv)), _full((de, dv)), _full((1, dv)),
                _full((1, dv)), _full((1, dv))]
    out_specs = [ebs, vbs]
    out_shape = [jax.ShapeDtypeStruct((e8, 128), jnp.float32),
                 jax.ShapeDtypeStruct((n, dv), jnp.float32)]
    args = [eo8, he8, h_v, agg0, agg1, cnt0, cnt1, m_avg, g_e, b_e,
            wvh, wvm, bv, g_v, b_v]
    if has_next:
        in_specs += [_full((128, 128)), _full((1, 128)),
                     _full((dv, de)), _full((dv, de))]
        out_specs += [ebs, sbs, sbs]
        out_shape += [jax.ShapeDtypeStruct((e8, 128), jnp.float32),
                      jax.ShapeDtypeStruct((n, de), jnp.float32),
                      jax.ShapeDtypeStruct((n, de), jnp.float32)]
        args += [nxt['ke'], nxt['be_t'], nxt['ws'], nxt['wd']]
    res = pl.pallas_call(
        body, grid=(grid,), in_specs=in_specs,
        out_specs=out_specs, out_shape=out_shape,
    )(*args)
    if has_next:
        return res
    return res[0], res[1], None, None, None


def _tile8(v):
    return jnp.tile(v, 8)[None, :]


def kernel(h_v, edge_index, h_e, params):
    n, dv = h_v.shape
    e, de = h_e.shape
    ei3 = edge_index.reshape(2, e // _CH, _CH)
    he8 = jnp.concatenate([h_e[j::8] for j in range(8)], axis=1)

    eye8 = jnp.eye(8, dtype=jnp.float32)
    m_avg = jnp.asarray(np.kron(np.eye(8, dtype=np.float32),
                                np.full((16, 16), 1.0 / 16, np.float32)))

    prep = []
    for p in params:
        prep.append({
            'ws': p['We'][:dv],
            'wd': p['We'][dv:2 * dv],
            'ke': jnp.kron(eye8, p['We'][2 * dv:]),
            'be_t': _tile8(p['be']),
            'wvh': p['Wv'][:dv],
            'wvm': p['Wv'][dv:],
            'bv': p['bv'][None, :],
            'g_v': p['g_v'][None, :],
            'b_v': p['b_v'][None, :],
            'g_e_t': _tile8(p['g_e']),
            'b_e_t': _tile8(p['b_e']),
        })

    nb = len(prep)
    a_tab, b_tab, c8 = _pre(h_v, he8, prep[0]['ws'], prep[0]['wd'],
                            prep[0]['ke'], prep[0]['be_t'])
    cnt0 = cnt1 = None
    for blk in range(nb):
        p = prep[blk]
        last = blk == nb - 1
        pn = None if last else prep[blk + 1]
        eo8, (agg0, agg1), cnts = _sc_edge_pass(
            a_tab, b_tab, c8, ei3, with_counts=(blk == 0))
        if cnts is not None:
            cnt0, cnt1 = cnts
        he8, h_v, c8, a_tab, b_tab = _post(
            eo8, he8, h_v, agg0, agg1, cnt0, cnt1, m_avg,
            p['g_e_t'], p['b_e_t'], p['wvh'], p['wvm'], p['bv'],
            p['g_v'], p['b_v'], nxt=pn)

    he_out = jnp.stack([he8[:, de * j:de * (j + 1)] for j in range(8)],
                       axis=1).reshape(e, de)
    return h_v, he_out


# speedup vs baseline: 1.8274x; 1.8274x over previous
import functools

import numpy as np
import jax
import jax.numpy as jnp
from jax import lax
from jax.experimental import pallas as pl
from jax.experimental.pallas import tpu as pltpu
from jax.experimental.pallas import tpu_sc as plsc

_NC = 2
_NS = 16
_L = 16
_CH = 128


_MC = 4
_ME = _MC * _CH


def _sc_edge_pass(a_tab, b_tab, c_t, ei3, with_counts):
    n = a_tab.shape[0]
    e = c_t.shape[1]
    nw = _NC * _NS
    nmac = e // _ME
    zr = 640
    npad = zr * _NS
    per_w = (nmac + nw - 1) // nw
    per_w += per_w % 2

    out_type = [
        jax.ShapeDtypeStruct((_L, e), jnp.float32),
        jax.ShapeDtypeStruct((_NC * npad, _L), jnp.float32),
    ]

    def bufset():
        return [
            pltpu.VMEM((2, _MC, _CH), jnp.int32),
            pltpu.VMEM((_ME, _L), jnp.float32),
            pltpu.VMEM((_ME, _L), jnp.float32),
            pltpu.VMEM((_L, _ME), jnp.float32),
            pltpu.VMEM((_ME, _L), jnp.float32),
            pltpu.VMEM((_L, _ME), jnp.float32),
            pltpu.SemaphoreType.DMA,
            pltpu.SemaphoreType.DMA,
            pltpu.SemaphoreType.DMA,
        ]

    scratch = bufset() + bufset() + [
        pltpu.VMEM((zr, _L), jnp.float32),
        pltpu.VMEM_SHARED((npad, _L), jnp.float32),
    ]
    if with_counts:
        out_type.append(jax.ShapeDtypeStruct((_NC * npad, _L), jnp.float32))
        scratch.append(pltpu.VMEM((_CH, _L), jnp.float32))
        scratch.append(pltpu.VMEM_SHARED((npad, _L), jnp.float32))

    mesh = plsc.VectorSubcoreMesh(core_axis_name="c", subcore_axis_name="s")

    @functools.partial(
        pl.kernel, out_type=tuple(out_type), mesh=mesh,
        scratch_types=scratch,
        compiler_params=pltpu.CompilerParams(use_tc_tiling_on_sc=False, needs_layout_passes=False))
    def sc_kernel(a_hbm, b_hbm, c_hbm, ei_hbm, eo_hbm, agg_hbm, *rest):
        if with_counts:
            cnt_hbm = rest[0]
            rest = rest[1:]
        s0 = rest[0:9]
        s1 = rest[9:18]
        zv, agg_sp = rest[18], rest[19]
        if with_counts:
            onesv, cnt_sp = rest[20], rest[21]
        cid = lax.axis_index("c")
        sid = lax.axis_index("s")
        wid = sid * _NC + cid
        rows16 = lax.iota(jnp.int32, _L)

        def issue_pre(m, S):
            idxb, _, _, cvt, _, _, sem_pre, _, _ = S
            mb = pl.multiple_of(m * _MC, _MC)
            pltpu.async_copy(ei_hbm.at[:, pl.ds(mb, _MC)], idxb, sem_pre)
            me = pl.multiple_of(m * _ME, _ME)
            pltpu.async_copy(c_hbm.at[:, pl.ds(me, _ME)], cvt, sem_pre)

        def wait_pre(S):
            idxb, _, _, cvt, _, _, sem_pre, _, _ = S
            pltpu.make_async_copy(ei_hbm.at[:, pl.ds(0, _MC)], idxb,
                                  sem_pre).wait()
            pltpu.make_async_copy(c_hbm.at[:, pl.ds(0, _ME)], cvt,
                                  sem_pre).wait()

        def issue_gath(S):
            idxb, av, bv, _, _, _, _, sem_g, _ = S
            for j in range(_MC):
                pltpu.async_copy(a_hbm.at[idxb.at[0, j]],
                                 av.at[pl.ds(j * _CH, _CH)], sem_g)
                pltpu.async_copy(b_hbm.at[idxb.at[1, j]],
                                 bv.at[pl.ds(j * _CH, _CH)], sem_g)

        def wait_gath(S):
            idxb, av, bv, _, _, _, _, sem_g, _ = S
            for j in range(_MC):
                pltpu.make_async_copy(a_hbm.at[idxb.at[0, j]],
                                      av.at[pl.ds(j * _CH, _CH)],
                                      sem_g).wait()
                pltpu.make_async_copy(b_hbm.at[idxb.at[1, j]],
                                      bv.at[pl.ds(j * _CH, _CH)],
                                      sem_g).wait()

        def drain_out(S):
            _, _, _, _, _, eovt, _, _, sem_out = S
            pltpu.make_async_copy(eovt, eo_hbm.at[:, pl.ds(0, _ME)],
                                  sem_out).wait()

        def run_macro(m, S):
            idxb, av, bv, cvt, eov, eovt, _, _, sem_out = S

            @pl.loop(0, _ME)
            def _edges(k):
                col = jnp.full((_L,), k, jnp.int32)
                cvec = plsc.load_gather(cvt, [rows16, col])
                v = av[k] + bv[k] + cvec
                eov[k] = v
                plsc.store_scatter(eovt, [rows16, col], v)

            me = pl.multiple_of(m * _ME, _ME)
            pltpu.async_copy(eovt, eo_hbm.at[:, pl.ds(me, _ME)], sem_out)
            for j in range(_MC):
                pltpu.sync_copy(eov.at[pl.ds(j * _CH, _CH)],
                                agg_sp.at[idxb.at[1, j]], add=True)
                if with_counts:
                    pltpu.sync_copy(onesv, cnt_sp.at[idxb.at[1, j]],
                                    add=True)

        @pl.loop(0, zr)
        def _zfill(j):
            zv[j] = jnp.zeros((_L,), jnp.float32)

        zoff = pl.multiple_of(sid * zr, zr)
        pltpu.sync_copy(zv, agg_sp.at[pl.ds(zoff, zr)])
        if with_counts:
            @pl.loop(0, _CH)
            def _ofill(j):
                onesv[j] = jnp.ones((_L,), jnp.float32)
            pltpu.sync_copy(zv, cnt_sp.at[pl.ds(zoff, zr)])
        plsc.subcore_barrier()

        issue_pre(wid, s0)
        issue_pre(wid + nw, s1)
        wait_pre(s0)
        issue_gath(s0)

        @pl.loop(0, per_w, step=2)
        def _pipe(k):
            for off, cur, nxt in ((0, s0, s1), (1, s1, s0)):
                kk = k + off
                m_cur = wid + kk * nw
                m_nxt = wid + (kk + 1) * nw
                m_pre = wid + (kk + 2) * nw

                @pl.when(m_cur < nmac)
                def _():
                    wait_gath(cur)

                    @pl.when(m_nxt < nmac)
                    def _():
                        wait_pre(nxt)
                        issue_gath(nxt)

                    @pl.when(kk >= 2)
                    def _():
                        drain_out(cur)

                    run_macro(m_cur, cur)

                    @pl.when(m_pre < nmac)
                    def _():
                        issue_pre(m_pre, cur)

        drain_out(s0)
        drain_out(s1)

        plsc.subcore_barrier()
        osl = pl.multiple_of(sid * zr, zr)
        ohb = pl.multiple_of(cid * npad + sid * zr, zr)
        pltpu.sync_copy(agg_sp.at[pl.ds(osl, zr)], agg_hbm.at[pl.ds(ohb, zr)])
        if with_counts:
            pltpu.sync_copy(cnt_sp.at[pl.ds(osl, zr)],
                            cnt_hbm.at[pl.ds(ohb, zr)])

    outs = sc_kernel(a_tab, b_tab, c_t, ei3)
    if with_counts:
        eot, agg, cnt = outs
        return eot, (agg[:n], agg[npad:npad + n]), (cnt[:n], cnt[npad:npad + n])
    eot, agg = outs
    return eot, (agg[:n], agg[npad:npad + n]), None


def _full(shape):
    return pl.BlockSpec(shape, lambda i: (0, 0))


def _pre(h_v, he_t, ws, wd, we_t, be_t, grid=25):
    n, dv = h_v.shape
    de = ws.shape[1]
    e = he_t.shape[1]
    bn = n // grid
    be_cols = e // grid

    def body(hv_ref, he_ref, ws_ref, wd_ref, wet_ref, bet_ref,
             a_ref, b_ref, c_ref):
        hv = hv_ref[...]
        a_ref[...] = jnp.dot(hv, ws_ref[...], preferred_element_type=jnp.float32)
        b_ref[...] = jnp.dot(hv, wd_ref[...], preferred_element_type=jnp.float32)
        c_ref[...] = (jnp.dot(wet_ref[...], he_ref[...],
                              preferred_element_type=jnp.float32) + bet_ref[...])

    tbs = pl.BlockSpec((de, be_cols), lambda i: (0, i))
    return pl.pallas_call(
        body,
        grid=(grid,),
        in_specs=[pl.BlockSpec((bn, dv), lambda i: (i, 0)), tbs,
                  _full((dv, de)), _full((dv, de)),
                  _full((de, de)), _full((de, 1))],
        out_specs=[pl.BlockSpec((bn, de), lambda i: (i, 0)),
                   pl.BlockSpec((bn, de), lambda i: (i, 0)), tbs],
        out_shape=[jax.ShapeDtypeStruct((n, de), jnp.float32),
                   jax.ShapeDtypeStruct((n, de), jnp.float32),
                   jax.ShapeDtypeStruct((de, e), jnp.float32)],
    )(h_v, he_t, ws, wd, we_t, be_t)


def _post(eo_t, he_t, h_v, agg0, agg1, cnt0, cnt1, m_t, g_e, b_e,
          wvh, wvm, bv, g_v, b_v, nxt=None, grid=25):
    de, e = eo_t.shape
    n, dv = h_v.shape
    be_cols = e // grid
    bn = n // grid
    has_next = nxt is not None

    def body(eo_ref, he_ref, hv_ref, a0_ref, a1_ref, c0_ref, c1_ref,
             m_ref, ge_ref, be_ref, wvh_ref, wvm_ref, bv_ref, gv_ref,
             bv2_ref, *rest):
        if has_next:
            (wen_ref, ben_ref, wsn_ref, wdn_ref,
             hen_ref, hvn_ref, cn_ref, an_ref, bn_ref) = rest
        else:
            hen_ref, hvn_ref = rest
        r = jnp.maximum(eo_ref[...], 0.0)
        mt = m_ref[...]
        mu = jnp.dot(mt, r, preferred_element_type=jnp.float32)
        q = r - mu
        var = jnp.dot(mt, q * q, preferred_element_type=jnp.float32)
        ln = q * lax.rsqrt(var + 1e-5) * ge_ref[...] + be_ref[...]
        hen = he_ref[...] + ln
        hen_ref[...] = hen
        aggt = a0_ref[...] + a1_ref[...]
        cntt = c0_ref[...] + c1_ref[...]
        mean = aggt / jnp.maximum(cntt, 1.0)
        hv = hv_ref[...]
        v = (jnp.dot(hv, wvh_ref[...], preferred_element_type=jnp.float32)
             + jnp.dot(mean, wvm_ref[...], preferred_element_type=jnp.float32)
             + bv_ref[...])
        v = jnp.maximum(v, 0.0)
        mu2 = jnp.mean(v, axis=-1, keepdims=True)
        q2 = v - mu2
        var2 = jnp.mean(q2 * q2, axis=-1, keepdims=True)
        ln2 = q2 * lax.rsqrt(var2 + 1e-5) * gv_ref[...] + bv2_ref[...]
        hvn = hv + ln2
        hvn_ref[...] = hvn
        if has_next:
            cn_ref[...] = (jnp.dot(wen_ref[...], hen,
                                   preferred_element_type=jnp.float32)
                           + ben_ref[...])
            an_ref[...] = jnp.dot(hvn, wsn_ref[...],
                                  preferred_element_type=jnp.float32)
            bn_ref[...] = jnp.dot(hvn, wdn_ref[...],
                                  preferred_element_type=jnp.float32)

    tbs = pl.BlockSpec((de, be_cols), lambda i: (0, i))
    vbs = pl.BlockSpec((bn, dv), lambda i: (i, 0))
    sbs = pl.BlockSpec((bn, de), lambda i: (i, 0))
    in_specs = [tbs, tbs, vbs, sbs, sbs, sbs, sbs,
                _full((de, de)), _full((de, 1)), _full((de, 1)),
                _full((dv, dv)), _full((de, dv)), _full((1, dv)),
                _full((1, dv)), _full((1, dv))]
    out_specs = [tbs, vbs]
    out_shape = [jax.ShapeDtypeStruct((de, e), jnp.float32),
                 jax.ShapeDtypeStruct((n, dv), jnp.float32)]
    args = [eo_t, he_t, h_v, agg0, agg1, cnt0, cnt1, m_t, g_e, b_e,
            wvh, wvm, bv, g_v, b_v]
    if has_next:
        in_specs += [_full((de, de)), _full((de, 1)),
                     _full((dv, de)), _full((dv, de))]
        out_specs += [tbs, sbs, sbs]
        out_shape += [jax.ShapeDtypeStruct((de, e), jnp.float32),
                      jax.ShapeDtypeStruct((n, de), jnp.float32),
                      jax.ShapeDtypeStruct((n, de), jnp.float32)]
        args += [nxt['we_t'], nxt['be_t'], nxt['ws'], nxt['wd']]
    res = pl.pallas_call(
        body, grid=(grid,), in_specs=in_specs,
        out_specs=out_specs, out_shape=out_shape,
    )(*args)
    if has_next:
        return res
    return res[0], res[1], None, None, None


def kernel(h_v, edge_index, h_e, params):
    n, dv = h_v.shape
    e, de = h_e.shape
    ei3 = edge_index.reshape(2, e // _CH, _CH)
    he_t = h_e.T

    m_t = jnp.full((de, de), 1.0 / de, jnp.float32)

    prep = []
    for p in params:
        prep.append({
            'ws': p['We'][:dv],
            'wd': p['We'][dv:2 * dv],
            'we_t': p['We'][2 * dv:].T,
            'be_t': p['be'][:, None],
            'wvh': p['Wv'][:dv],
            'wvm': p['Wv'][dv:],
            'bv': p['bv'][None, :],
            'g_v': p['g_v'][None, :],
            'b_v': p['b_v'][None, :],
            'g_e_t': p['g_e'][:, None],
            'b_e_t': p['b_e'][:, None],
        })

    nb = len(prep)
    a_tab, b_tab, c_t = _pre(h_v, he_t, prep[0]['ws'], prep[0]['wd'],
                             prep[0]['we_t'], prep[0]['be_t'])
    cnt0 = cnt1 = None
    for blk in range(nb):
        p = prep[blk]
        last = blk == nb - 1
        pn = None if last else prep[blk + 1]
        eo_t, (agg0, agg1), cnts = _sc_edge_pass(
            a_tab, b_tab, c_t, ei3, with_counts=(blk == 0))
        if cnts is not None:
            cnt0, cnt1 = cnts
        he_t, h_v, c_t, a_tab, b_tab = _post(
            eo_t, he_t, h_v, agg0, agg1, cnt0, cnt1, m_t,
            p['g_e_t'], p['b_e_t'], p['wvh'], p['wvm'], p['bv'],
            p['g_v'], p['b_v'], nxt=pn)

    return h_v, he_t.T


# speedup vs baseline: 2.7355x; 1.4969x over previous
import functools

import numpy as np
import jax
import jax.numpy as jnp
from jax import lax
from jax.experimental import pallas as pl
from jax.experimental.pallas import tpu as pltpu
from jax.experimental.pallas import tpu_sc as plsc

_NC = 2
_NS = 16
_L = 16
_CH = 128


_MC = 4
_ME = _MC * _CH


def _sc_edge_pass(a_tab, b_tab, c_t, ei3, with_counts):
    n = a_tab.shape[0]
    e = c_t.shape[1]
    nw = _NC * _NS
    nmac = e // _ME
    zr = 640
    npad = zr * _NS
    per_w = (nmac + nw - 1) // nw
    per_w += per_w % 2

    out_type = [
        jax.ShapeDtypeStruct((_L, e), jnp.float32),
        jax.ShapeDtypeStruct((_NC * npad, _L), jnp.float32),
    ]

    def bufset():
        return [
            pltpu.VMEM((2, _MC, _CH), jnp.int32),
            pltpu.VMEM((_ME, _L), jnp.float32),
            pltpu.VMEM((_ME, _L), jnp.float32),
            pltpu.VMEM((_L, _ME + 1), jnp.float32),
            pltpu.VMEM((_ME, _L), jnp.float32),
            pltpu.VMEM((_L, _ME + 1), jnp.float32),
            pltpu.SemaphoreType.DMA,
            pltpu.SemaphoreType.DMA,
            pltpu.SemaphoreType.DMA,
        ]

    scratch = bufset() + bufset() + [
        pltpu.VMEM((zr, _L), jnp.float32),
        pltpu.VMEM_SHARED((npad, _L), jnp.float32),
    ]
    if with_counts:
        out_type.append(jax.ShapeDtypeStruct((_NC * npad, _L), jnp.float32))
        scratch.append(pltpu.VMEM((_CH, _L), jnp.float32))
        scratch.append(pltpu.VMEM_SHARED((npad, _L), jnp.float32))

    mesh = plsc.VectorSubcoreMesh(core_axis_name="c", subcore_axis_name="s")

    @functools.partial(
        pl.kernel, out_type=tuple(out_type), mesh=mesh,
        scratch_types=scratch,
        compiler_params=pltpu.CompilerParams(use_tc_tiling_on_sc=False, needs_layout_passes=False))
    def sc_kernel(a_hbm, b_hbm, c_hbm, ei_hbm, eo_hbm, agg_hbm, *rest):
        if with_counts:
            cnt_hbm = rest[0]
            rest = rest[1:]
        s0 = rest[0:9]
        s1 = rest[9:18]
        zv, agg_sp = rest[18], rest[19]
        if with_counts:
            onesv, cnt_sp = rest[20], rest[21]
        cid = lax.axis_index("c")
        sid = lax.axis_index("s")
        wid = sid * _NC + cid
        rows16 = lax.iota(jnp.int32, _L)

        def issue_pre(m, S):
            idxb, _, _, cvt, _, _, sem_pre, _, _ = S
            mb = pl.multiple_of(m * _MC, _MC)
            pltpu.async_copy(ei_hbm.at[:, pl.ds(mb, _MC)], idxb, sem_pre)
            me = pl.multiple_of(m * _ME, _ME)
            pltpu.async_copy(c_hbm.at[:, pl.ds(me, _ME)],
                             cvt.at[:, pl.ds(0, _ME)], sem_pre)

        def wait_pre(S):
            idxb, _, _, cvt, _, _, sem_pre, _, _ = S
            pltpu.make_async_copy(ei_hbm.at[:, pl.ds(0, _MC)], idxb,
                                  sem_pre).wait()
            pltpu.make_async_copy(c_hbm.at[:, pl.ds(0, _ME)],
                                  cvt.at[:, pl.ds(0, _ME)], sem_pre).wait()

        def issue_gath(S):
            idxb, av, bv, _, _, _, _, sem_g, _ = S
            for j in range(_MC):
                pltpu.async_copy(a_hbm.at[idxb.at[0, j]],
                                 av.at[pl.ds(j * _CH, _CH)], sem_g)
                pltpu.async_copy(b_hbm.at[idxb.at[1, j]],
                                 bv.at[pl.ds(j * _CH, _CH)], sem_g)

        def wait_gath(S):
            idxb, av, bv, _, _, _, _, sem_g, _ = S
            for j in range(_MC):
                pltpu.make_async_copy(a_hbm.at[idxb.at[0, j]],
                                      av.at[pl.ds(j * _CH, _CH)],
                                      sem_g).wait()
                pltpu.make_async_copy(b_hbm.at[idxb.at[1, j]],
                                      bv.at[pl.ds(j * _CH, _CH)],
                                      sem_g).wait()

        def drain_out(S):
            _, _, _, _, _, eovt, _, _, sem_out = S
            pltpu.make_async_copy(eovt.at[:, pl.ds(0, _ME)],
                                  eo_hbm.at[:, pl.ds(0, _ME)],
                                  sem_out).wait()

        def run_macro(m, S):
            idxb, av, bv, cvt, eov, eovt, _, _, sem_out = S

            @pl.loop(0, _ME, unroll=4)
            def _edges(k):
                col = jnp.full((_L,), k, jnp.int32)
                cvec = plsc.load_gather(cvt, [rows16, col])
                v = av[k] + bv[k] + cvec
                eov[k] = v
                plsc.store_scatter(eovt, [rows16, col], v)

            me = pl.multiple_of(m * _ME, _ME)
            pltpu.async_copy(eovt.at[:, pl.ds(0, _ME)],
                             eo_hbm.at[:, pl.ds(me, _ME)], sem_out)
            for j in range(_MC):
                pltpu.sync_copy(eov.at[pl.ds(j * _CH, _CH)],
                                agg_sp.at[idxb.at[1, j]], add=True)
                if with_counts:
                    pltpu.sync_copy(onesv, cnt_sp.at[idxb.at[1, j]],
                                    add=True)

        @pl.loop(0, zr)
        def _zfill(j):
            zv[j] = jnp.zeros((_L,), jnp.float32)

        zoff = pl.multiple_of(sid * zr, zr)
        pltpu.sync_copy(zv, agg_sp.at[pl.ds(zoff, zr)])
        if with_counts:
            @pl.loop(0, _CH)
            def _ofill(j):
                onesv[j] = jnp.ones((_L,), jnp.float32)
            pltpu.sync_copy(zv, cnt_sp.at[pl.ds(zoff, zr)])
        plsc.subcore_barrier()

        issue_pre(wid, s0)
        issue_pre(wid + nw, s1)
        wait_pre(s0)
        issue_gath(s0)

        @pl.loop(0, per_w, step=2)
        def _pipe(k):
            for off, cur, nxt in ((0, s0, s1), (1, s1, s0)):
                kk = k + off
                m_cur = wid + kk * nw
                m_nxt = wid + (kk + 1) * nw
                m_pre = wid + (kk + 2) * nw

                @pl.when(m_cur < nmac)
                def _():
                    wait_gath(cur)

                    @pl.when(m_nxt < nmac)
                    def _():
                        wait_pre(nxt)
                        issue_gath(nxt)

                    @pl.when(kk >= 2)
                    def _():
                        drain_out(cur)

                    run_macro(m_cur, cur)

                    @pl.when(m_pre < nmac)
                    def _():
                        issue_pre(m_pre, cur)

        drain_out(s0)
        drain_out(s1)

        plsc.subcore_barrier()
        osl = pl.multiple_of(sid * zr, zr)
        ohb = pl.multiple_of(cid * npad + sid * zr, zr)
        pltpu.sync_copy(agg_sp.at[pl.ds(osl, zr)], agg_hbm.at[pl.ds(ohb, zr)])
        if with_counts:
            pltpu.sync_copy(cnt_sp.at[pl.ds(osl, zr)],
                            cnt_hbm.at[pl.ds(ohb, zr)])

    outs = sc_kernel(a_tab, b_tab, c_t, ei3)
    if with_counts:
        eot, agg, cnt = outs
        return eot, (agg[:n], agg[npad:npad + n]), (cnt[:n], cnt[npad:npad + n])
    eot, agg = outs
    return eot, (agg[:n], agg[npad:npad + n]), None


def _full(shape):
    return pl.BlockSpec(shape, lambda i: (0, 0))


def _pre(h_v, he_t, ws, wd, we_t, be_t, grid=25):
    n, dv = h_v.shape
    de = ws.shape[1]
    e = he_t.shape[1]
    bn = n // grid
    be_cols = e // grid

    def body(hv_ref, he_ref, ws_ref, wd_ref, wet_ref, bet_ref,
             a_ref, b_ref, c_ref):
        hv = hv_ref[...]
        a_ref[...] = jnp.dot(hv, ws_ref[...], preferred_element_type=jnp.float32)
        b_ref[...] = jnp.dot(hv, wd_ref[...], preferred_element_type=jnp.float32)
        c_ref[...] = (jnp.dot(wet_ref[...], he_ref[...],
                              preferred_element_type=jnp.float32) + bet_ref[...])

    tbs = pl.BlockSpec((de, be_cols), lambda i: (0, i))
    return pl.pallas_call(
        body,
        grid=(grid,),
        in_specs=[pl.BlockSpec((bn, dv), lambda i: (i, 0)), tbs,
                  _full((dv, de)), _full((dv, de)),
                  _full((de, de)), _full((de, 1))],
        out_specs=[pl.BlockSpec((bn, de), lambda i: (i, 0)),
                   pl.BlockSpec((bn, de), lambda i: (i, 0)), tbs],
        out_shape=[jax.ShapeDtypeStruct((n, de), jnp.float32),
                   jax.ShapeDtypeStruct((n, de), jnp.float32),
                   jax.ShapeDtypeStruct((de, e), jnp.float32)],
    )(h_v, he_t, ws, wd, we_t, be_t)


def _post(eo_t, he_t, h_v, agg0, agg1, cnt0, cnt1, m_t, g_e, b_e,
          wvh, wvm, bv, g_v, b_v, nxt=None, grid=25):
    de, e = eo_t.shape
    n, dv = h_v.shape
    be_cols = e // grid
    bn = n // grid
    has_next = nxt is not None

    def body(eo_ref, he_ref, hv_ref, a0_ref, a1_ref, c0_ref, c1_ref,
             m_ref, ge_ref, be_ref, wvh_ref, wvm_ref, bv_ref, gv_ref,
             bv2_ref, *rest):
        if has_next:
            (wen_ref, ben_ref, wsn_ref, wdn_ref,
             hen_ref, hvn_ref, cn_ref, an_ref, bn_ref) = rest
        else:
            hen_ref, hvn_ref = rest
        r = jnp.maximum(eo_ref[...], 0.0)
        mt = m_ref[...]
        mu = jnp.dot(mt, r, preferred_element_type=jnp.float32)
        q = r - mu
        var = jnp.dot(mt, q * q, preferred_element_type=jnp.float32)
        ln = q * lax.rsqrt(var + 1e-5) * ge_ref[...] + be_ref[...]
        hen = he_ref[...] + ln
        hen_ref[...] = hen
        aggt = a0_ref[...] + a1_ref[...]
        cntt = c0_ref[...] + c1_ref[...]
        mean = aggt / jnp.maximum(cntt, 1.0)
        hv = hv_ref[...]
        v = (jnp.dot(hv, wvh_ref[...], preferred_element_type=jnp.float32)
             + jnp.dot(mean, wvm_ref[...], preferred_element_type=jnp.float32)
             + bv_ref[...])
        v = jnp.maximum(v, 0.0)
        mu2 = jnp.mean(v, axis=-1, keepdims=True)
        q2 = v - mu2
        var2 = jnp.mean(q2 * q2, axis=-1, keepdims=True)
        ln2 = q2 * lax.rsqrt(var2 + 1e-5) * gv_ref[...] + bv2_ref[...]
        hvn = hv + ln2
        hvn_ref[...] = hvn
        if has_next:
            cn_ref[...] = (jnp.dot(wen_ref[...], hen,
                                   preferred_element_type=jnp.float32)
                           + ben_ref[...])
            an_ref[...] = jnp.dot(hvn, wsn_ref[...],
                                  preferred_element_type=jnp.float32)
            bn_ref[...] = jnp.dot(hvn, wdn_ref[...],
                                  preferred_element_type=jnp.float32)

    tbs = pl.BlockSpec((de, be_cols), lambda i: (0, i))
    vbs = pl.BlockSpec((bn, dv), lambda i: (i, 0))
    sbs = pl.BlockSpec((bn, de), lambda i: (i, 0))
    in_specs = [tbs, tbs, vbs, sbs, sbs, sbs, sbs,
                _full((de, de)), _full((de, 1)), _full((de, 1)),
                _full((dv, dv)), _full((de, dv)), _full((1, dv)),
                _full((1, dv)), _full((1, dv))]
    out_specs = [tbs, vbs]
    out_shape = [jax.ShapeDtypeStruct((de, e), jnp.float32),
                 jax.ShapeDtypeStruct((n, dv), jnp.float32)]
    args = [eo_t, he_t, h_v, agg0, agg1, cnt0, cnt1, m_t, g_e, b_e,
            wvh, wvm, bv, g_v, b_v]
    if has_next:
        in_specs += [_full((de, de)), _full((de, 1)),
                     _full((dv, de)), _full((dv, de))]
        out_specs += [tbs, sbs, sbs]
        out_shape += [jax.ShapeDtypeStruct((de, e), jnp.float32),
                      jax.ShapeDtypeStruct((n, de), jnp.float32),
                      jax.ShapeDtypeStruct((n, de), jnp.float32)]
        args += [nxt['we_t'], nxt['be_t'], nxt['ws'], nxt['wd']]
    res = pl.pallas_call(
        body, grid=(grid,), in_specs=in_specs,
        out_specs=out_specs, out_shape=out_shape,
    )(*args)
    if has_next:
        return res
    return res[0], res[1], None, None, None


def kernel(h_v, edge_index, h_e, params):
    n, dv = h_v.shape
    e, de = h_e.shape
    ei3 = edge_index.reshape(2, e // _CH, _CH)
    he_t = h_e.T

    m_t = jnp.full((de, de), 1.0 / de, jnp.float32)

    prep = []
    for p in params:
        prep.append({
            'ws': p['We'][:dv],
            'wd': p['We'][dv:2 * dv],
            'we_t': p['We'][2 * dv:].T,
            'be_t': p['be'][:, None],
            'wvh': p['Wv'][:dv],
            'wvm': p['Wv'][dv:],
            'bv': p['bv'][None, :],
            'g_v': p['g_v'][None, :],
            'b_v': p['b_v'][None, :],
            'g_e_t': p['g_e'][:, None],
            'b_e_t': p['b_e'][:, None],
        })

    nb = len(prep)
    a_tab, b_tab, c_t = _pre(h_v, he_t, prep[0]['ws'], prep[0]['wd'],
                             prep[0]['we_t'], prep[0]['be_t'])
    cnt0 = cnt1 = None
    for blk in range(nb):
        p = prep[blk]
        last = blk == nb - 1
        pn = None if last else prep[blk + 1]
        eo_t, (agg0, agg1), cnts = _sc_edge_pass(
            a_tab, b_tab, c_t, ei3, with_counts=(blk == 0))
        if cnts is not None:
            cnt0, cnt1 = cnts
        he_t, h_v, c_t, a_tab, b_tab = _post(
            eo_t, he_t, h_v, agg0, agg1, cnt0, cnt1, m_t,
            p['g_e_t'], p['b_e_t'], p['wvh'], p['wvm'], p['bv'],
            p['g_v'], p['b_v'], nxt=pn)

    return h_v, he_t.T


# speedup vs baseline: 2.8441x; 1.0397x over previous
import functools

import numpy as np
import jax
import jax.numpy as jnp
from jax import lax
from jax.experimental import pallas as pl
from jax.experimental.pallas import tpu as pltpu
from jax.experimental.pallas import tpu_sc as plsc

_NC = 2
_NS = 16
_L = 16
_CH = 128


_MC = 4
_ME = _MC * _CH


def _sc_edge_pass(a_tab, b_tab, c_t, ei3, with_counts):
    n = a_tab.shape[0]
    e = c_t.shape[1]
    nw = _NC * _NS
    nmac = e // _ME
    zr = 640
    npad = zr * _NS
    per_w = (nmac + nw - 1) // nw
    per_w += per_w % 2

    out_type = [
        jax.ShapeDtypeStruct((_L, e), jnp.float32),
        jax.ShapeDtypeStruct((_NC * npad, _L), jnp.float32),
    ]

    def bufset():
        return [
            pltpu.VMEM((2, _MC, _CH), jnp.int32),
            pltpu.VMEM((_ME, _L), jnp.float32),
            pltpu.VMEM((_ME, _L), jnp.float32),
            pltpu.VMEM((_L, _ME + 1), jnp.float32),
            pltpu.VMEM((_ME, _L), jnp.float32),
            pltpu.VMEM((_L, _ME + 1), jnp.float32),
            pltpu.SemaphoreType.DMA,
            pltpu.SemaphoreType.DMA,
            pltpu.SemaphoreType.DMA,
        ]

    scratch = bufset() + bufset() + [
        pltpu.VMEM((zr, _L), jnp.float32),
        pltpu.VMEM_SHARED((npad, _L), jnp.float32),
    ]
    if with_counts:
        out_type.append(jax.ShapeDtypeStruct((_NC * npad, _L), jnp.float32))
        scratch.append(pltpu.VMEM((_CH, _L), jnp.float32))
        scratch.append(pltpu.VMEM_SHARED((npad, _L), jnp.float32))

    mesh = plsc.VectorSubcoreMesh(core_axis_name="c", subcore_axis_name="s")

    @functools.partial(
        pl.kernel, out_type=tuple(out_type), mesh=mesh,
        scratch_types=scratch,
        compiler_params=pltpu.CompilerParams(use_tc_tiling_on_sc=False, needs_layout_passes=False))
    def sc_kernel(a_hbm, b_hbm, c_hbm, ei_hbm, eo_hbm, agg_hbm, *rest):
        if with_counts:
            cnt_hbm = rest[0]
            rest = rest[1:]
        s0 = rest[0:9]
        s1 = rest[9:18]
        zv, agg_sp = rest[18], rest[19]
        if with_counts:
            onesv, cnt_sp = rest[20], rest[21]
        cid = lax.axis_index("c")
        sid = lax.axis_index("s")
        wid = sid * _NC + cid
        rows16 = lax.iota(jnp.int32, _L)

        def issue_pre(m, S):
            idxb, _, _, cvt, _, _, sem_pre, _, _ = S
            mb = pl.multiple_of(m * _MC, _MC)
            pltpu.async_copy(ei_hbm.at[:, pl.ds(mb, _MC)], idxb, sem_pre)
            me = pl.multiple_of(m * _ME, _ME)
            pltpu.async_copy(c_hbm.at[:, pl.ds(me, _ME)],
                             cvt.at[:, pl.ds(0, _ME)], sem_pre)

        def wait_pre(S):
            idxb, _, _, cvt, _, _, sem_pre, _, _ = S
            pltpu.make_async_copy(ei_hbm.at[:, pl.ds(0, _MC)], idxb,
                                  sem_pre).wait()
            pltpu.make_async_copy(c_hbm.at[:, pl.ds(0, _ME)],
                                  cvt.at[:, pl.ds(0, _ME)], sem_pre).wait()

        def issue_gath(S):
            idxb, av, bv, _, _, _, _, sem_g, _ = S
            for j in range(_MC):
                pltpu.async_copy(a_hbm.at[idxb.at[0, j]],
                                 av.at[pl.ds(j * _CH, _CH)], sem_g)
                pltpu.async_copy(b_hbm.at[idxb.at[1, j]],
                                 bv.at[pl.ds(j * _CH, _CH)], sem_g)

        def wait_gath(S):
            idxb, av, bv, _, _, _, _, sem_g, _ = S
            for j in range(_MC):
                pltpu.make_async_copy(a_hbm.at[idxb.at[0, j]],
                                      av.at[pl.ds(j * _CH, _CH)],
                                      sem_g).wait()
                pltpu.make_async_copy(b_hbm.at[idxb.at[1, j]],
                                      bv.at[pl.ds(j * _CH, _CH)],
                                      sem_g).wait()

        def drain_out(S):
            _, _, _, _, _, eovt, _, _, sem_out = S
            pltpu.make_async_copy(eovt.at[:, pl.ds(0, _ME)],
                                  eo_hbm.at[:, pl.ds(0, _ME)],
                                  sem_out).wait()

        def run_macro(m, S):
            idxb, av, bv, cvt, eov, eovt, _, _, sem_out = S

            @pl.loop(0, _ME, unroll=4)
            def _edges(k):
                col = jnp.full((_L,), k, jnp.int32)
                cvec = plsc.load_gather(cvt, [rows16, col])
                v = av[k] + bv[k] + cvec
                eov[k] = v
                plsc.store_scatter(eovt, [rows16, col], v)

            me = pl.multiple_of(m * _ME, _ME)
            pltpu.async_copy(eovt.at[:, pl.ds(0, _ME)],
                             eo_hbm.at[:, pl.ds(me, _ME)], sem_out)
            for j in range(_MC):
                pltpu.sync_copy(eov.at[pl.ds(j * _CH, _CH)],
                                agg_sp.at[idxb.at[1, j]], add=True)
                if with_counts:
                    pltpu.sync_copy(onesv, cnt_sp.at[idxb.at[1, j]],
                                    add=True)

        @pl.loop(0, zr)
        def _zfill(j):
            zv[j] = jnp.zeros((_L,), jnp.float32)

        zoff = pl.multiple_of(sid * zr, zr)
        pltpu.sync_copy(zv, agg_sp.at[pl.ds(zoff, zr)])
        if with_counts:
            @pl.loop(0, _CH)
            def _ofill(j):
                onesv[j] = jnp.ones((_L,), jnp.float32)
            pltpu.sync_copy(zv, cnt_sp.at[pl.ds(zoff, zr)])
        plsc.subcore_barrier()

        issue_pre(wid, s0)
        issue_pre(wid + nw, s1)
        wait_pre(s0)
        issue_gath(s0)

        @pl.loop(0, per_w, step=2)
        def _pipe(k):
            for off, cur, nxt in ((0, s0, s1), (1, s1, s0)):
                kk = k + off
                m_cur = wid + kk * nw
                m_nxt = wid + (kk + 1) * nw
                m_pre = wid + (kk + 2) * nw

                @pl.when(m_cur < nmac)
                def _():
                    wait_gath(cur)

                    @pl.when(m_nxt < nmac)
                    def _():
                        wait_pre(nxt)
                        issue_gath(nxt)

                    @pl.when(kk >= 2)
                    def _():
                        drain_out(cur)

                    run_macro(m_cur, cur)

                    @pl.when(m_pre < nmac)
                    def _():
                        issue_pre(m_pre, cur)

        drain_out(s0)
        drain_out(s1)

        plsc.subcore_barrier()
        osl = pl.multiple_of(sid * zr, zr)
        ohb = pl.multiple_of(cid * npad + sid * zr, zr)
        pltpu.sync_copy(agg_sp.at[pl.ds(osl, zr)], agg_hbm.at[pl.ds(ohb, zr)])
        if with_counts:
            pltpu.sync_copy(cnt_sp.at[pl.ds(osl, zr)],
                            cnt_hbm.at[pl.ds(ohb, zr)])

    outs = sc_kernel(a_tab, b_tab, c_t, ei3)
    if with_counts:
        eot, agg, cnt = outs
        return eot, (agg[:n], agg[npad:npad + n]), (cnt[:n], cnt[npad:npad + n])
    eot, agg = outs
    return eot, (agg[:n], agg[npad:npad + n]), None


def _full(shape):
    return pl.BlockSpec(shape, lambda i: (0, 0))


def _pre(h_v, he_t, ws, wd, we_t, be_t, grid=10):
    n, dv = h_v.shape
    de = ws.shape[1]
    e = he_t.shape[1]
    bn = n // grid
    be_cols = e // grid

    def body(hv_ref, he_ref, ws_ref, wd_ref, wet_ref, bet_ref,
             a_ref, b_ref, c_ref):
        hv = hv_ref[...]
        a_ref[...] = jnp.dot(hv, ws_ref[...], preferred_element_type=jnp.float32)
        b_ref[...] = jnp.dot(hv, wd_ref[...], preferred_element_type=jnp.float32)
        c_ref[...] = (jnp.dot(wet_ref[...], he_ref[...],
                              preferred_element_type=jnp.float32) + bet_ref[...])

    tbs = pl.BlockSpec((de, be_cols), lambda i: (0, i))
    return pl.pallas_call(
        body,
        grid=(grid,),
        in_specs=[pl.BlockSpec((bn, dv), lambda i: (i, 0)), tbs,
                  _full((dv, de)), _full((dv, de)),
                  _full((de, de)), _full((de, 1))],
        out_specs=[pl.BlockSpec((bn, de), lambda i: (i, 0)),
                   pl.BlockSpec((bn, de), lambda i: (i, 0)), tbs],
        out_shape=[jax.ShapeDtypeStruct((n, de), jnp.float32),
                   jax.ShapeDtypeStruct((n, de), jnp.float32),
                   jax.ShapeDtypeStruct((de, e), jnp.float32)],
    )(h_v, he_t, ws, wd, we_t, be_t)


def _post(eo_t, he_t, h_v, agg0, agg1, cnt0, cnt1, m_t, g_e, b_e,
          wvh, wvm, bv, g_v, b_v, nxt=None, grid=10):
    de, e = eo_t.shape
    n, dv = h_v.shape
    be_cols = e // grid
    bn = n // grid
    has_next = nxt is not None

    def body(eo_ref, he_ref, hv_ref, a0_ref, a1_ref, c0_ref, c1_ref,
             m_ref, ge_ref, be_ref, wvh_ref, wvm_ref, bv_ref, gv_ref,
             bv2_ref, *rest):
        if has_next:
            (wen_ref, ben_ref, wsn_ref, wdn_ref,
             hen_ref, hvn_ref, cn_ref, an_ref, bn_ref) = rest
        else:
            hen_ref, hvn_ref = rest
        r = jnp.maximum(eo_ref[...], 0.0)
        mt = m_ref[...]
        mu = jnp.dot(mt, r, preferred_element_type=jnp.float32)
        q = r - mu
        var = jnp.dot(mt, q * q, preferred_element_type=jnp.float32)
        ln = q * lax.rsqrt(var + 1e-5) * ge_ref[...] + be_ref[...]
        hen = he_ref[...] + ln
        hen_ref[...] = hen
        aggt = a0_ref[...] + a1_ref[...]
        cntt = c0_ref[...] + c1_ref[...]
        mean = aggt / jnp.maximum(cntt, 1.0)
        hv = hv_ref[...]
        v = (jnp.dot(hv, wvh_ref[...], preferred_element_type=jnp.float32)
             + jnp.dot(mean, wvm_ref[...], preferred_element_type=jnp.float32)
             + bv_ref[...])
        v = jnp.maximum(v, 0.0)
        mu2 = jnp.mean(v, axis=-1, keepdims=True)
        q2 = v - mu2
        var2 = jnp.mean(q2 * q2, axis=-1, keepdims=True)
        ln2 = q2 * lax.rsqrt(var2 + 1e-5) * gv_ref[...] + bv2_ref[...]
        hvn = hv + ln2
        hvn_ref[...] = hvn
        if has_next:
            cn_ref[...] = (jnp.dot(wen_ref[...], hen,
                                   preferred_element_type=jnp.float32)
                           + ben_ref[...])
            an_ref[...] = jnp.dot(hvn, wsn_ref[...],
                                  preferred_element_type=jnp.float32)
            bn_ref[...] = jnp.dot(hvn, wdn_ref[...],
                                  preferred_element_type=jnp.float32)

    tbs = pl.BlockSpec((de, be_cols), lambda i: (0, i))
    vbs = pl.BlockSpec((bn, dv), lambda i: (i, 0))
    sbs = pl.BlockSpec((bn, de), lambda i: (i, 0))
    in_specs = [tbs, tbs, vbs, sbs, sbs, sbs, sbs,
                _full((de, de)), _full((de, 1)), _full((de, 1)),
                _full((dv, dv)), _full((de, dv)), _full((1, dv)),
                _full((1, dv)), _full((1, dv))]
    out_specs = [tbs, vbs]
    out_shape = [jax.ShapeDtypeStruct((de, e), jnp.float32),
                 jax.ShapeDtypeStruct((n, dv), jnp.float32)]
    args = [eo_t, he_t, h_v, agg0, agg1, cnt0, cnt1, m_t, g_e, b_e,
            wvh, wvm, bv, g_v, b_v]
    if has_next:
        in_specs += [_full((de, de)), _full((de, 1)),
                     _full((dv, de)), _full((dv, de))]
        out_specs += [tbs, sbs, sbs]
        out_shape += [jax.ShapeDtypeStruct((de, e), jnp.float32),
                      jax.ShapeDtypeStruct((n, de), jnp.float32),
                      jax.ShapeDtypeStruct((n, de), jnp.float32)]
        args += [nxt['we_t'], nxt['be_t'], nxt['ws'], nxt['wd']]
    res = pl.pallas_call(
        body, grid=(grid,), in_specs=in_specs,
        out_specs=out_specs, out_shape=out_shape,
    )(*args)
    if has_next:
        return res
    return res[0], res[1], None, None, None


def kernel(h_v, edge_index, h_e, params):
    n, dv = h_v.shape
    e, de = h_e.shape
    ei3 = edge_index.reshape(2, e // _CH, _CH)
    he_t = h_e.T

    m_t = jnp.full((de, de), 1.0 / de, jnp.float32)

    prep = []
    for p in params:
        prep.append({
            'ws': p['We'][:dv],
            'wd': p['We'][dv:2 * dv],
            'we_t': p['We'][2 * dv:].T,
            'be_t': p['be'][:, None],
            'wvh': p['Wv'][:dv],
            'wvm': p['Wv'][dv:],
            'bv': p['bv'][None, :],
            'g_v': p['g_v'][None, :],
            'b_v': p['b_v'][None, :],
            'g_e_t': p['g_e'][:, None],
            'b_e_t': p['b_e'][:, None],
        })

    nb = len(prep)
    a_tab, b_tab, c_t = _pre(h_v, he_t, prep[0]['ws'], prep[0]['wd'],
                             prep[0]['we_t'], prep[0]['be_t'])
    cnt0 = cnt1 = None
    for blk in range(nb):
        p = prep[blk]
        last = blk == nb - 1
        pn = None if last else prep[blk + 1]
        eo_t, (agg0, agg1), cnts = _sc_edge_pass(
            a_tab, b_tab, c_t, ei3, with_counts=(blk == 0))
        if cnts is not None:
            cnt0, cnt1 = cnts
        he_t, h_v, c_t, a_tab, b_tab = _post(
            eo_t, he_t, h_v, agg0, agg1, cnt0, cnt1, m_t,
            p['g_e_t'], p['b_e_t'], p['wvh'], p['wvm'], p['bv'],
            p['g_v'], p['b_v'], nxt=pn)

    return h_v, he_t.T
